# baseline (speedup 1.0000x reference)
import functools
import math

import jax
import jax.numpy as jnp
from jax.experimental import pallas as pl
from jax.experimental.pallas import tpu as pltpu

B = 1; S = 2048; D = 768; H = 12; DH = 64; FFN = 3072; E = 8; TOPK = 2
TM = 256
P = 6144
NT = P // TM
TS = 256
QT = 256
NEG = -1e9


def _ln_block(xb, g, b):
    m = jnp.mean(xb, axis=1, keepdims=True)
    v = jnp.mean((xb - m) ** 2, axis=1, keepdims=True)
    return (xb - m) / jnp.sqrt(v + 1e-5) * g + b


def _qkv_kernel(x_ref, g_ref, b_ref, wq_ref, bq_ref, wk_ref, bk_ref,
                wv_ref, bv_ref, sin_ref, cos_ref, q_ref, k_ref, v_ref):
    h = _ln_block(x_ref[...], g_ref[...], b_ref[...])
    qf = jnp.dot(h, wq_ref[...], preferred_element_type=jnp.float32) + bq_ref[...]
    kf = jnp.dot(h, wk_ref[...], preferred_element_type=jnp.float32) + bk_ref[...]
    vf = jnp.dot(h, wv_ref[...], preferred_element_type=jnp.float32) + bv_ref[...]
    sin = sin_ref[...]
    cos = cos_ref[...]
    for hh in range(H):
        qh = qf[:, hh * DH:(hh + 1) * DH]
        kh = kf[:, hh * DH:(hh + 1) * DH]
        q1, q2 = qh[:, :DH // 2], qh[:, DH // 2:]
        k1, k2 = kh[:, :DH // 2], kh[:, DH // 2:]
        q_ref[hh, :, :] = jnp.concatenate(
            [cos * q1 - sin * q2, sin * q1 + cos * q2], axis=1)
        k_ref[hh, :, :] = jnp.concatenate(
            [cos * k1 - sin * k2, sin * k1 + cos * k2], axis=1)
        v_ref[hh, :, :] = vf[:, hh * DH:(hh + 1) * DH]


SH = S // 2


def _attn_chunk(q, k, v, i, j0):
    sk = k.shape[0]
    s = jax.lax.dot_general(q, k, (((1,), (1,)), ((), ())),
                            preferred_element_type=jnp.float32)
    s = s * jnp.float32(1.0 / math.sqrt(DH))
    row = jax.lax.broadcasted_iota(jnp.int32, (QT, sk), 0) + i * QT
    col = jax.lax.broadcasted_iota(jnp.int32, (QT, sk), 1) + j0
    s = jnp.where(col <= row, s, NEG)
    m = jnp.max(s, axis=1, keepdims=True)
    p = jnp.exp(s - m)
    l = jnp.sum(p, axis=1, keepdims=True)
    pv = jnp.dot(p, v, preferred_element_type=jnp.float32)
    return m, l, pv


KC = 512


def _attn_kernel(q_ref, k_ref, v_ref, o_ref):
    i = pl.program_id(1)
    q = q_ref[0]
    qper = KC // QT

    for nc in range(1, S // KC + 1):
        @pl.when((i // qper) == nc - 1)
        def _(nc=nc):
            parts = [_attn_chunk(q, k_ref[0, c * KC:(c + 1) * KC, :],
                                 v_ref[0, c * KC:(c + 1) * KC, :], i, c * KC)
                     for c in range(nc)]
            if nc == 1:
                m, l, pv = parts[0]
                o_ref[0] = pv / l
            else:
                m = parts[0][0]
                for mm, _, _ in parts[1:]:
                    m = jnp.maximum(m, mm)
                lt = jnp.zeros_like(parts[0][1])
                pt = jnp.zeros_like(parts[0][2])
                for mm, ll, vv in parts:
                    cc = jnp.exp(mm - m)
                    lt = lt + ll * cc
                    pt = pt + vv * cc
                o_ref[0] = pt / lt


def _post_kernel(ao_ref, x_ref, wo_ref, bo_ref, g_ref, b_ref, wr_ref,
                 x1_ref, h2_ref, ti_ref, tw_ref, eu_ref, zl_ref):
    i = pl.program_id(0)
    ao = jnp.concatenate([ao_ref[hh] for hh in range(H)], axis=1)
    x1 = x_ref[...] + jnp.dot(ao, wo_ref[...],
                              preferred_element_type=jnp.float32) + bo_ref[...]
    x1_ref[...] = x1
    h2 = _ln_block(x1, g_ref[...], b_ref[...])
    h2_ref[...] = h2.astype(jnp.bfloat16)
    logits = jnp.dot(h2, wr_ref[...], preferred_element_type=jnp.float32)
    lm = jnp.max(logits, axis=1, keepdims=True)
    ex = jnp.exp(logits - lm)
    se = jnp.sum(ex, axis=1, keepdims=True)
    probs = ex / se
    lse = lm + jnp.log(se)
    iota8 = jax.lax.broadcasted_iota(jnp.int32, (TS, E), 1)
    v1 = jnp.max(probs, axis=1, keepdims=True)
    i1 = jnp.min(jnp.where(probs == v1, iota8, E), axis=1, keepdims=True)
    p2 = jnp.where(iota8 == i1, jnp.float32(-1.0), probs)
    v2 = jnp.max(p2, axis=1, keepdims=True)
    i2 = jnp.min(jnp.where(p2 == v2, iota8, E), axis=1, keepdims=True)
    wsum = v1 + v2 + 1e-8
    ti_ref[...] = jnp.concatenate([i1, i2], axis=1)
    tw_ref[...] = jnp.concatenate([v1 / wsum, v2 / wsum], axis=1)

    @pl.when(i == 0)
    def _():
        eu_ref[...] = jnp.zeros_like(eu_ref)
        zl_ref[...] = jnp.zeros_like(zl_ref)

    eu_ref[...] += jnp.sum(probs, axis=0, keepdims=True)
    zl_ref[0:1, 0:1] += jnp.sum(lse * lse).reshape(1, 1)


def _rank_kernel(ti_ref, rank_ref, cnt_ref):
    CH = 512
    tri = (jax.lax.broadcasted_iota(jnp.int32, (CH, CH), 0)
           > jax.lax.broadcasted_iota(jnp.int32, (CH, CH), 1)).astype(jnp.float32)
    iota8 = jax.lax.broadcasted_iota(jnp.int32, (CH, E), 1)
    carry = jnp.zeros((1, E), jnp.float32)
    for c in range(S * TOPK // CH):
        e = ti_ref[c * CH:(c + 1) * CH, :]
        ohc = (e == iota8).astype(jnp.float32)
        excl = jnp.dot(tri, ohc, preferred_element_type=jnp.float32) + carry
        rank_ref[c * CH:(c + 1) * CH, :] = jnp.sum(
            excl * ohc, axis=1, keepdims=True).astype(jnp.int32)
        carry = carry + jnp.sum(ohc, axis=0, keepdims=True)
    cnt_ref[...] = carry.astype(jnp.int32)


def _moe_kernel(te_ref, st_ref, h2_ref, w1_ref, b1_ref, w2_ref, b2_ref,
                w_ref, o_ref):
    st = st_ref[...]
    col = jax.lax.broadcasted_iota(jnp.int32, (TM, S), 1)
    onehot = (col == st).astype(jnp.bfloat16)
    xs = jnp.dot(onehot, h2_ref[...],
                 preferred_element_type=jnp.float32).astype(jnp.bfloat16)
    FC = 768
    out = jnp.zeros((TM, D), jnp.float32) + b2_ref[0]
    for c in range(FFN // FC):
        hmid = jnp.dot(xs, w1_ref[0][:, c * FC:(c + 1) * FC],
                       preferred_element_type=jnp.float32) + b1_ref[0][:, c * FC:(c + 1) * FC]
        hmid = hmid * 0.5 * (1.0 + jax.lax.erf(hmid * jnp.float32(1.0 / math.sqrt(2.0))))
        out = out + jnp.dot(hmid.astype(jnp.bfloat16),
                            w2_ref[0][c * FC:(c + 1) * FC, :],
                            preferred_element_type=jnp.float32)
    o_ref[...] = (out * w_ref[...]).astype(jnp.bfloat16)


def _combine_kernel(p0_ref, p1_ref, x1_ref, ffn_ref, y_ref):
    col = jax.lax.broadcasted_iota(jnp.int32, (TS, P), 1)
    oh = ((col == p0_ref[...]) | (col == p1_ref[...])).astype(jnp.bfloat16)
    y_ref[...] = x1_ref[...] + jnp.dot(oh, ffn_ref[...],
                                       preferred_element_type=jnp.float32)


def kernel(x, ln1_g, ln1_b, ln2_g, ln2_b, Wq, bq, Wk, bk, Wv, bv, Wo, bo,
           Wr, W1, b1, W2, b2):
    f32 = jnp.float32
    xs2 = x.reshape(S, D)
    row1 = lambda t: t.reshape(1, -1)

    pos_t = jnp.arange(S, dtype=f32)
    div_t = jnp.exp(jnp.arange(0, DH, 2, dtype=f32)
                    * -(jnp.log(10000.0) / DH))
    ang_t = pos_t[:, None] * div_t[None, :]
    sin_t = jnp.sin(ang_t)
    cos_t = jnp.cos(ang_t)

    q, k, v = pl.pallas_call(
        _qkv_kernel,
        grid=(S // TS,),
        in_specs=[
            pl.BlockSpec((TS, D), lambda i: (i, 0)),
            pl.BlockSpec((1, D), lambda i: (0, 0)),
            pl.BlockSpec((1, D), lambda i: (0, 0)),
            pl.BlockSpec((D, D), lambda i: (0, 0)),
            pl.BlockSpec((1, D), lambda i: (0, 0)),
            pl.BlockSpec((D, D), lambda i: (0, 0)),
            pl.BlockSpec((1, D), lambda i: (0, 0)),
            pl.BlockSpec((D, D), lambda i: (0, 0)),
            pl.BlockSpec((1, D), lambda i: (0, 0)),
            pl.BlockSpec((TS, DH // 2), lambda i: (i, 0)),
            pl.BlockSpec((TS, DH // 2), lambda i: (i, 0)),
        ],
        out_specs=[
            pl.BlockSpec((H, TS, DH), lambda i: (0, i, 0)),
            pl.BlockSpec((H, TS, DH), lambda i: (0, i, 0)),
            pl.BlockSpec((H, TS, DH), lambda i: (0, i, 0)),
        ],
        out_shape=[jax.ShapeDtypeStruct((H, S, DH), f32)] * 3,
    )(xs2, row1(ln1_g), row1(ln1_b), Wq, row1(bq), Wk, row1(bk), Wv, row1(bv),
      sin_t, cos_t)

    ao = pl.pallas_call(
        _attn_kernel,
        grid=(H, S // QT),
        in_specs=[
            pl.BlockSpec((1, QT, DH), lambda h, i: (h, i, 0)),
            pl.BlockSpec((1, S, DH), lambda h, i: (h, 0, 0)),
            pl.BlockSpec((1, S, DH), lambda h, i: (h, 0, 0)),
        ],
        out_specs=pl.BlockSpec((1, QT, DH), lambda h, i: (h, i, 0)),
        out_shape=jax.ShapeDtypeStruct((H, S, DH), f32),
    )(q, k, v)

    x1, h2, ti, tw, eu_sum, zl_sum = pl.pallas_call(
        _post_kernel,
        grid=(S // TS,),
        in_specs=[
            pl.BlockSpec((H, TS, DH), lambda i: (0, i, 0)),
            pl.BlockSpec((TS, D), lambda i: (i, 0)),
            pl.BlockSpec((D, D), lambda i: (0, 0)),
            pl.BlockSpec((1, D), lambda i: (0, 0)),
            pl.BlockSpec((1, D), lambda i: (0, 0)),
            pl.BlockSpec((1, D), lambda i: (0, 0)),
            pl.BlockSpec((D, E), lambda i: (0, 0)),
        ],
        out_specs=[
            pl.BlockSpec((TS, D), lambda i: (i, 0)),
            pl.BlockSpec((TS, D), lambda i: (i, 0)),
            pl.BlockSpec((TS, TOPK), lambda i: (i, 0)),
            pl.BlockSpec((TS, TOPK), lambda i: (i, 0)),
            pl.BlockSpec((1, E), lambda i: (0, 0)),
            pl.BlockSpec((1, E), lambda i: (0, 0)),
        ],
        out_shape=[
            jax.ShapeDtypeStruct((S, D), f32),
            jax.ShapeDtypeStruct((S, D), jnp.bfloat16),
            jax.ShapeDtypeStruct((S, TOPK), jnp.int32),
            jax.ShapeDtypeStruct((S, TOPK), f32),
            jax.ShapeDtypeStruct((1, E), f32),
            jax.ShapeDtypeStruct((1, E), f32),
        ],
    )(ao, xs2, Wo, row1(bo), row1(ln2_g), row1(ln2_b), Wr)

    i32 = jnp.int32
    flat_e = ti.reshape(-1)
    flat_w = tw.reshape(-1)
    rank2, cnt = pl.pallas_call(
        _rank_kernel,
        grid=(1,),
        in_specs=[pl.BlockSpec((S * TOPK, 1), lambda i: (0, 0))],
        out_specs=[pl.BlockSpec((S * TOPK, 1), lambda i: (0, 0)),
                   pl.BlockSpec((1, E), lambda i: (0, 0))],
        out_shape=[jax.ShapeDtypeStruct((S * TOPK, 1), i32),
                   jax.ShapeDtypeStruct((1, E), i32)],
    )(flat_e.reshape(S * TOPK, 1))
    counts = cnt[0]
    padded = ((counts + TM - 1) // TM) * TM
    cum_p = jnp.cumsum(padded)
    gstart = cum_p - padded
    dest = (jnp.take(gstart, flat_e) + rank2[:, 0]).astype(i32)
    pos_slot = dest
    jj = jnp.arange(S * TOPK, dtype=i32)
    src_tok = jnp.zeros((P,), i32).at[dest].set(jj // TOPK)
    w_pad = jnp.zeros((P,), f32).at[dest].set(flat_w).reshape(P, 1)
    tile_start = jnp.arange(NT, dtype=i32) * TM
    te = jnp.minimum(jnp.searchsorted(cum_p, tile_start, side='right'),
                     E - 1).astype(i32)

    ffn_out = pl.pallas_call(
        _moe_kernel,
        grid_spec=pltpu.PrefetchScalarGridSpec(
            num_scalar_prefetch=1,
            grid=(NT,),
            in_specs=[
                pl.BlockSpec((TM, 1), lambda i, te: (i, 0)),
                pl.BlockSpec((S, D), lambda i, te: (0, 0)),
                pl.BlockSpec((1, D, FFN), lambda i, te: (te[i], 0, 0)),
                pl.BlockSpec((1, 1, FFN), lambda i, te: (te[i], 0, 0)),
                pl.BlockSpec((1, FFN, D), lambda i, te: (te[i], 0, 0)),
                pl.BlockSpec((1, 1, D), lambda i, te: (te[i], 0, 0)),
                pl.BlockSpec((TM, 1), lambda i, te: (i, 0)),
            ],
            out_specs=pl.BlockSpec((TM, D), lambda i, te: (i, 0)),
        ),
        out_shape=jax.ShapeDtypeStruct((P, D), jnp.bfloat16),
    )(te, src_tok.reshape(P, 1), h2, W1.astype(jnp.bfloat16),
      b1.reshape(E, 1, FFN), W2.astype(jnp.bfloat16), b2.reshape(E, 1, D),
      w_pad)

    pos2 = pos_slot.reshape(S, TOPK)
    y = pl.pallas_call(
        _combine_kernel,
        grid=(S // TS,),
        in_specs=[
            pl.BlockSpec((TS, 1), lambda i: (i, 0)),
            pl.BlockSpec((TS, 1), lambda i: (i, 0)),
            pl.BlockSpec((TS, D), lambda i: (i, 0)),
            pl.BlockSpec((P, D), lambda i: (0, 0)),
        ],
        out_specs=pl.BlockSpec((TS, D), lambda i: (i, 0)),
        out_shape=jax.ShapeDtypeStruct((S, D), f32),
    )(pos2[:, 0:1], pos2[:, 1:2], x1, ffn_out)

    eu = eu_sum[0] / jnp.float32(S)
    mu = jnp.mean(eu)
    lb = jnp.mean((eu - mu) ** 2) / (mu * mu + 1e-8) * float(E) * 0.01
    zl = zl_sum[0, 0] / jnp.float32(S) * 0.001
    return y.reshape(B, S, D), lb, zl


# speedup vs baseline: 1.0039x; 1.0039x over previous
import functools
import math

import jax
import jax.numpy as jnp
from jax.experimental import pallas as pl
from jax.experimental.pallas import tpu as pltpu

B = 1; S = 2048; D = 768; H = 12; DH = 64; FFN = 3072; E = 8; TOPK = 2
TM = 256
P = 6144
NT = P // TM
TS = 256
QT = 256
NEG = -1e9


def _ln_block(xb, g, b):
    m = jnp.mean(xb, axis=1, keepdims=True)
    v = jnp.mean((xb - m) ** 2, axis=1, keepdims=True)
    return (xb - m) / jnp.sqrt(v + 1e-5) * g + b


def _qkv_kernel(x_ref, g_ref, b_ref, wq_ref, bq_ref, wk_ref, bk_ref,
                wv_ref, bv_ref, sin_ref, cos_ref, q_ref, k_ref, v_ref):
    h = _ln_block(x_ref[...], g_ref[...], b_ref[...])
    qf = jnp.dot(h, wq_ref[...], preferred_element_type=jnp.float32) + bq_ref[...]
    kf = jnp.dot(h, wk_ref[...], preferred_element_type=jnp.float32) + bk_ref[...]
    vf = jnp.dot(h, wv_ref[...], preferred_element_type=jnp.float32) + bv_ref[...]
    sin = sin_ref[...]
    cos = cos_ref[...]
    for hh in range(H):
        qh = qf[:, hh * DH:(hh + 1) * DH]
        kh = kf[:, hh * DH:(hh + 1) * DH]
        q1, q2 = qh[:, :DH // 2], qh[:, DH // 2:]
        k1, k2 = kh[:, :DH // 2], kh[:, DH // 2:]
        q_ref[hh, :, :] = jnp.concatenate(
            [cos * q1 - sin * q2, sin * q1 + cos * q2], axis=1)
        k_ref[hh, :, :] = jnp.concatenate(
            [cos * k1 - sin * k2, sin * k1 + cos * k2], axis=1)
        v_ref[hh, :, :] = vf[:, hh * DH:(hh + 1) * DH]


SH = S // 2


def _attn_chunk(q, k, v, i, j0):
    sk = k.shape[0]
    s = jax.lax.dot_general(q, k, (((1,), (1,)), ((), ())),
                            preferred_element_type=jnp.float32)
    s = s * jnp.float32(1.0 / math.sqrt(DH))
    row = jax.lax.broadcasted_iota(jnp.int32, (QT, sk), 0) + i * QT
    col = jax.lax.broadcasted_iota(jnp.int32, (QT, sk), 1) + j0
    s = jnp.where(col <= row, s, NEG)
    m = jnp.max(s, axis=1, keepdims=True)
    p = jnp.exp(s - m)
    l = jnp.sum(p, axis=1, keepdims=True)
    pv = jnp.dot(p, v, preferred_element_type=jnp.float32)
    return m, l, pv


KC = 512


def _attn_kernel(q_ref, k_ref, v_ref, o_ref):
    i = pl.program_id(1)
    q = q_ref[0]
    qper = KC // QT

    for nc in range(1, S // KC + 1):
        @pl.when((i // qper) == nc - 1)
        def _(nc=nc):
            parts = [_attn_chunk(q, k_ref[0, c * KC:(c + 1) * KC, :],
                                 v_ref[0, c * KC:(c + 1) * KC, :], i, c * KC)
                     for c in range(nc)]
            if nc == 1:
                m, l, pv = parts[0]
                o_ref[0] = pv / l
            else:
                m = parts[0][0]
                for mm, _, _ in parts[1:]:
                    m = jnp.maximum(m, mm)
                lt = jnp.zeros_like(parts[0][1])
                pt = jnp.zeros_like(parts[0][2])
                for mm, ll, vv in parts:
                    cc = jnp.exp(mm - m)
                    lt = lt + ll * cc
                    pt = pt + vv * cc
                o_ref[0] = pt / lt


def _post_kernel(ao_ref, x_ref, wo_ref, bo_ref, g_ref, b_ref, wr_ref,
                 x1_ref, h2_ref, ti_ref, tw_ref, eu_ref, zl_ref):
    i = pl.program_id(0)
    ao = jnp.concatenate([ao_ref[hh] for hh in range(H)], axis=1)
    x1 = x_ref[...] + jnp.dot(ao, wo_ref[...],
                              preferred_element_type=jnp.float32) + bo_ref[...]
    x1_ref[...] = x1
    h2 = _ln_block(x1, g_ref[...], b_ref[...])
    h2_ref[...] = h2.astype(jnp.bfloat16)
    logits = jnp.dot(h2, wr_ref[...], preferred_element_type=jnp.float32)
    lm = jnp.max(logits, axis=1, keepdims=True)
    ex = jnp.exp(logits - lm)
    se = jnp.sum(ex, axis=1, keepdims=True)
    probs = ex / se
    lse = lm + jnp.log(se)
    iota8 = jax.lax.broadcasted_iota(jnp.int32, (TS, E), 1)
    v1 = jnp.max(probs, axis=1, keepdims=True)
    i1 = jnp.min(jnp.where(probs == v1, iota8, E), axis=1, keepdims=True)
    p2 = jnp.where(iota8 == i1, jnp.float32(-1.0), probs)
    v2 = jnp.max(p2, axis=1, keepdims=True)
    i2 = jnp.min(jnp.where(p2 == v2, iota8, E), axis=1, keepdims=True)
    wsum = v1 + v2 + 1e-8
    ti_ref[...] = jnp.concatenate([i1, i2], axis=1)
    tw_ref[...] = jnp.concatenate([v1 / wsum, v2 / wsum], axis=1)

    @pl.when(i == 0)
    def _():
        eu_ref[...] = jnp.zeros_like(eu_ref)
        zl_ref[...] = jnp.zeros_like(zl_ref)

    eu_ref[...] += jnp.sum(probs, axis=0, keepdims=True)
    zl_ref[0:1, 0:1] += jnp.sum(lse * lse).reshape(1, 1)


def _moe_kernel(te_ref, st_ref, h2_ref, w1_ref, b1_ref, w2_ref, b2_ref,
                w_ref, o_ref):
    st = st_ref[...]
    col = jax.lax.broadcasted_iota(jnp.int32, (TM, S), 1)
    onehot = (col == st).astype(jnp.bfloat16)
    xs = jnp.dot(onehot, h2_ref[...],
                 preferred_element_type=jnp.float32).astype(jnp.bfloat16)
    FC = 1536
    out = jnp.zeros((TM, D), jnp.float32) + b2_ref[0]
    for c in range(FFN // FC):
        hmid = jnp.dot(xs, w1_ref[0][:, c * FC:(c + 1) * FC],
                       preferred_element_type=jnp.float32) + b1_ref[0][:, c * FC:(c + 1) * FC]
        hmid = hmid * 0.5 * (1.0 + jax.lax.erf(hmid * jnp.float32(1.0 / math.sqrt(2.0))))
        out = out + jnp.dot(hmid.astype(jnp.bfloat16),
                            w2_ref[0][c * FC:(c + 1) * FC, :],
                            preferred_element_type=jnp.float32)
    o_ref[...] = (out * w_ref[...]).astype(jnp.bfloat16)


def _combine_kernel(p0_ref, p1_ref, x1_ref, ffn_ref, y_ref):
    col = jax.lax.broadcasted_iota(jnp.int32, (TS, P), 1)
    oh = ((col == p0_ref[...]) | (col == p1_ref[...])).astype(jnp.bfloat16)
    y_ref[...] = x1_ref[...] + jnp.dot(oh, ffn_ref[...],
                                       preferred_element_type=jnp.float32)


def kernel(x, ln1_g, ln1_b, ln2_g, ln2_b, Wq, bq, Wk, bk, Wv, bv, Wo, bo,
           Wr, W1, b1, W2, b2):
    f32 = jnp.float32
    xs2 = x.reshape(S, D)
    row1 = lambda t: t.reshape(1, -1)

    pos_t = jnp.arange(S, dtype=f32)
    div_t = jnp.exp(jnp.arange(0, DH, 2, dtype=f32)
                    * -(jnp.log(10000.0) / DH))
    ang_t = pos_t[:, None] * div_t[None, :]
    sin_t = jnp.sin(ang_t)
    cos_t = jnp.cos(ang_t)

    q, k, v = pl.pallas_call(
        _qkv_kernel,
        grid=(S // TS,),
        in_specs=[
            pl.BlockSpec((TS, D), lambda i: (i, 0)),
            pl.BlockSpec((1, D), lambda i: (0, 0)),
            pl.BlockSpec((1, D), lambda i: (0, 0)),
            pl.BlockSpec((D, D), lambda i: (0, 0)),
            pl.BlockSpec((1, D), lambda i: (0, 0)),
            pl.BlockSpec((D, D), lambda i: (0, 0)),
            pl.BlockSpec((1, D), lambda i: (0, 0)),
            pl.BlockSpec((D, D), lambda i: (0, 0)),
            pl.BlockSpec((1, D), lambda i: (0, 0)),
            pl.BlockSpec((TS, DH // 2), lambda i: (i, 0)),
            pl.BlockSpec((TS, DH // 2), lambda i: (i, 0)),
        ],
        out_specs=[
            pl.BlockSpec((H, TS, DH), lambda i: (0, i, 0)),
            pl.BlockSpec((H, TS, DH), lambda i: (0, i, 0)),
            pl.BlockSpec((H, TS, DH), lambda i: (0, i, 0)),
        ],
        out_shape=[jax.ShapeDtypeStruct((H, S, DH), f32)] * 3,
    )(xs2, row1(ln1_g), row1(ln1_b), Wq, row1(bq), Wk, row1(bk), Wv, row1(bv),
      sin_t, cos_t)

    ao = pl.pallas_call(
        _attn_kernel,
        grid=(H, S // QT),
        in_specs=[
            pl.BlockSpec((1, QT, DH), lambda h, i: (h, i, 0)),
            pl.BlockSpec((1, S, DH), lambda h, i: (h, 0, 0)),
            pl.BlockSpec((1, S, DH), lambda h, i: (h, 0, 0)),
        ],
        out_specs=pl.BlockSpec((1, QT, DH), lambda h, i: (h, i, 0)),
        out_shape=jax.ShapeDtypeStruct((H, S, DH), f32),
    )(q, k, v)

    x1, h2, ti, tw, eu_sum, zl_sum = pl.pallas_call(
        _post_kernel,
        grid=(S // TS,),
        in_specs=[
            pl.BlockSpec((H, TS, DH), lambda i: (0, i, 0)),
            pl.BlockSpec((TS, D), lambda i: (i, 0)),
            pl.BlockSpec((D, D), lambda i: (0, 0)),
            pl.BlockSpec((1, D), lambda i: (0, 0)),
            pl.BlockSpec((1, D), lambda i: (0, 0)),
            pl.BlockSpec((1, D), lambda i: (0, 0)),
            pl.BlockSpec((D, E), lambda i: (0, 0)),
        ],
        out_specs=[
            pl.BlockSpec((TS, D), lambda i: (i, 0)),
            pl.BlockSpec((TS, D), lambda i: (i, 0)),
            pl.BlockSpec((TS, TOPK), lambda i: (i, 0)),
            pl.BlockSpec((TS, TOPK), lambda i: (i, 0)),
            pl.BlockSpec((1, E), lambda i: (0, 0)),
            pl.BlockSpec((1, E), lambda i: (0, 0)),
        ],
        out_shape=[
            jax.ShapeDtypeStruct((S, D), f32),
            jax.ShapeDtypeStruct((S, D), jnp.bfloat16),
            jax.ShapeDtypeStruct((S, TOPK), jnp.int32),
            jax.ShapeDtypeStruct((S, TOPK), f32),
            jax.ShapeDtypeStruct((1, E), f32),
            jax.ShapeDtypeStruct((1, E), f32),
        ],
    )(ao, xs2, Wo, row1(bo), row1(ln2_g), row1(ln2_b), Wr)

    i32 = jnp.int32
    flat_e = ti.reshape(-1)
    flat_w = tw.reshape(-1)
    oh = (flat_e[:, None] == jnp.arange(E, dtype=i32)[None, :]).astype(i32)
    csum = jnp.cumsum(oh, axis=0)
    counts = csum[-1]
    rank = jnp.sum(csum * oh, axis=1) - 1
    padded = ((counts + TM - 1) // TM) * TM
    cum_p = jnp.cumsum(padded)
    gstart = cum_p - padded
    dest = (jnp.sum(oh * gstart[None, :], axis=1) + rank).astype(i32)
    pos_slot = dest
    jj = jnp.arange(S * TOPK, dtype=i32)
    src_tok = jnp.zeros((P,), i32).at[dest].set(jj // TOPK)
    w_pad = jnp.zeros((P,), f32).at[dest].set(flat_w).reshape(P, 1)
    tile_start = jnp.arange(NT, dtype=i32) * TM
    te = jnp.minimum(jnp.searchsorted(cum_p, tile_start, side='right'),
                     E - 1).astype(i32)

    ffn_out = pl.pallas_call(
        _moe_kernel,
        grid_spec=pltpu.PrefetchScalarGridSpec(
            num_scalar_prefetch=1,
            grid=(NT,),
            in_specs=[
                pl.BlockSpec((TM, 1), lambda i, te: (i, 0)),
                pl.BlockSpec((S, D), lambda i, te: (0, 0)),
                pl.BlockSpec((1, D, FFN), lambda i, te: (te[i], 0, 0)),
                pl.BlockSpec((1, 1, FFN), lambda i, te: (te[i], 0, 0)),
                pl.BlockSpec((1, FFN, D), lambda i, te: (te[i], 0, 0)),
                pl.BlockSpec((1, 1, D), lambda i, te: (te[i], 0, 0)),
                pl.BlockSpec((TM, 1), lambda i, te: (i, 0)),
            ],
            out_specs=pl.BlockSpec((TM, D), lambda i, te: (i, 0)),
        ),
        out_shape=jax.ShapeDtypeStruct((P, D), jnp.bfloat16),
    )(te, src_tok.reshape(P, 1), h2, W1.astype(jnp.bfloat16),
      b1.reshape(E, 1, FFN), W2.astype(jnp.bfloat16), b2.reshape(E, 1, D),
      w_pad)

    pos2 = pos_slot.reshape(S, TOPK)
    y = pl.pallas_call(
        _combine_kernel,
        grid=(S // TS,),
        in_specs=[
            pl.BlockSpec((TS, 1), lambda i: (i, 0)),
            pl.BlockSpec((TS, 1), lambda i: (i, 0)),
            pl.BlockSpec((TS, D), lambda i: (i, 0)),
            pl.BlockSpec((P, D), lambda i: (0, 0)),
        ],
        out_specs=pl.BlockSpec((TS, D), lambda i: (i, 0)),
        out_shape=jax.ShapeDtypeStruct((S, D), f32),
    )(pos2[:, 0:1], pos2[:, 1:2], x1, ffn_out)

    eu = eu_sum[0] / jnp.float32(S)
    mu = jnp.mean(eu)
    lb = jnp.mean((eu - mu) ** 2) / (mu * mu + 1e-8) * float(E) * 0.01
    zl = zl_sum[0, 0] / jnp.float32(S) * 0.001
    return y.reshape(B, S, D), lb, zl


# speedup vs baseline: 1.0400x; 1.0360x over previous
import functools
import math

import jax
import jax.numpy as jnp
from jax.experimental import pallas as pl
from jax.experimental.pallas import tpu as pltpu

B = 1; S = 2048; D = 768; H = 12; DH = 64; FFN = 3072; E = 8; TOPK = 2
TM = 256
P = 6144
NT = P // TM
TS = 256
QT = 256
NEG = -1e9


def _ln_block(xb, g, b):
    m = jnp.mean(xb, axis=1, keepdims=True)
    v = jnp.mean((xb - m) ** 2, axis=1, keepdims=True)
    return (xb - m) / jnp.sqrt(v + 1e-5) * g + b


def _qkv_kernel(x_ref, g_ref, b_ref, wq_ref, bq_ref, wk_ref, bk_ref,
                wv_ref, bv_ref, sin_ref, cos_ref, q_ref, k_ref, v_ref):
    h = _ln_block(x_ref[...], g_ref[...], b_ref[...])
    qf = jnp.dot(h, wq_ref[...], preferred_element_type=jnp.float32) + bq_ref[...]
    kf = jnp.dot(h, wk_ref[...], preferred_element_type=jnp.float32) + bk_ref[...]
    vf = jnp.dot(h, wv_ref[...], preferred_element_type=jnp.float32) + bv_ref[...]
    sin = sin_ref[...]
    cos = cos_ref[...]
    for hh in range(H):
        qh = qf[:, hh * DH:(hh + 1) * DH]
        kh = kf[:, hh * DH:(hh + 1) * DH]
        q1, q2 = qh[:, :DH // 2], qh[:, DH // 2:]
        k1, k2 = kh[:, :DH // 2], kh[:, DH // 2:]
        q_ref[hh, :, :] = jnp.concatenate(
            [cos * q1 - sin * q2, sin * q1 + cos * q2], axis=1)
        k_ref[hh, :, :] = jnp.concatenate(
            [cos * k1 - sin * k2, sin * k1 + cos * k2], axis=1)
        v_ref[hh, :, :] = vf[:, hh * DH:(hh + 1) * DH]


SH = S // 2


def _attn_chunk(q, k, v, i, j0):
    sk = k.shape[0]
    s = jax.lax.dot_general(q, k, (((1,), (1,)), ((), ())),
                            preferred_element_type=jnp.float32)
    s = s * jnp.float32(1.0 / math.sqrt(DH))
    row = jax.lax.broadcasted_iota(jnp.int32, (QT, sk), 0) + i * QT
    col = jax.lax.broadcasted_iota(jnp.int32, (QT, sk), 1) + j0
    s = jnp.where(col <= row, s, NEG)
    m = jnp.max(s, axis=1, keepdims=True)
    p = jnp.exp(s - m)
    l = jnp.sum(p, axis=1, keepdims=True)
    pv = jnp.dot(p, v, preferred_element_type=jnp.float32)
    return m, l, pv


KC = 256


def _attn_kernel(q_ref, k_ref, v_ref, o_ref):
    i = pl.program_id(1)
    q = q_ref[0]
    qper = KC // QT

    for nc in range(1, S // KC + 1):
        @pl.when((i // qper) == nc - 1)
        def _(nc=nc):
            parts = [_attn_chunk(q, k_ref[0, c * KC:(c + 1) * KC, :],
                                 v_ref[0, c * KC:(c + 1) * KC, :], i, c * KC)
                     for c in range(nc)]
            if nc == 1:
                m, l, pv = parts[0]
                o_ref[0] = pv / l
            else:
                m = parts[0][0]
                for mm, _, _ in parts[1:]:
                    m = jnp.maximum(m, mm)
                lt = jnp.zeros_like(parts[0][1])
                pt = jnp.zeros_like(parts[0][2])
                for mm, ll, vv in parts:
                    cc = jnp.exp(mm - m)
                    lt = lt + ll * cc
                    pt = pt + vv * cc
                o_ref[0] = pt / lt


def _post_kernel(ao_ref, x_ref, wo_ref, bo_ref, g_ref, b_ref, wr_ref,
                 x1_ref, h2_ref, ti_ref, tw_ref, eu_ref, zl_ref):
    i = pl.program_id(0)
    ao = jnp.concatenate([ao_ref[hh] for hh in range(H)], axis=1)
    x1 = x_ref[...] + jnp.dot(ao, wo_ref[...],
                              preferred_element_type=jnp.float32) + bo_ref[...]
    x1_ref[...] = x1
    h2 = _ln_block(x1, g_ref[...], b_ref[...])
    h2_ref[...] = h2.astype(jnp.bfloat16)
    logits = jnp.dot(h2, wr_ref[...], preferred_element_type=jnp.float32)
    lm = jnp.max(logits, axis=1, keepdims=True)
    ex = jnp.exp(logits - lm)
    se = jnp.sum(ex, axis=1, keepdims=True)
    probs = ex / se
    lse = lm + jnp.log(se)
    iota8 = jax.lax.broadcasted_iota(jnp.int32, (TS, E), 1)
    v1 = jnp.max(probs, axis=1, keepdims=True)
    i1 = jnp.min(jnp.where(probs == v1, iota8, E), axis=1, keepdims=True)
    p2 = jnp.where(iota8 == i1, jnp.float32(-1.0), probs)
    v2 = jnp.max(p2, axis=1, keepdims=True)
    i2 = jnp.min(jnp.where(p2 == v2, iota8, E), axis=1, keepdims=True)
    wsum = v1 + v2 + 1e-8
    ti_ref[...] = jnp.concatenate([i1, i2], axis=1)
    tw_ref[...] = jnp.concatenate([v1 / wsum, v2 / wsum], axis=1)

    @pl.when(i == 0)
    def _():
        eu_ref[...] = jnp.zeros_like(eu_ref)
        zl_ref[...] = jnp.zeros_like(zl_ref)

    eu_ref[...] += jnp.sum(probs, axis=0, keepdims=True)
    zl_ref[0:1, 0:1] += jnp.sum(lse * lse).reshape(1, 1)


def _moe_kernel(te_ref, st_ref, h2_ref, w1_ref, b1_ref, w2_ref, b2_ref,
                w_ref, o_ref):
    st = st_ref[...]
    col = jax.lax.broadcasted_iota(jnp.int32, (TM, S), 1)
    onehot = (col == st).astype(jnp.bfloat16)
    xs = jnp.dot(onehot, h2_ref[...],
                 preferred_element_type=jnp.float32).astype(jnp.bfloat16)
    FC = 768
    out = jnp.zeros((TM, D), jnp.float32) + b2_ref[0]
    for c in range(FFN // FC):
        hmid = jnp.dot(xs, w1_ref[0][:, c * FC:(c + 1) * FC],
                       preferred_element_type=jnp.float32) + b1_ref[0][:, c * FC:(c + 1) * FC]
        hmid = hmid * 0.5 * (1.0 + jax.lax.erf(hmid * jnp.float32(1.0 / math.sqrt(2.0))))
        out = out + jnp.dot(hmid.astype(jnp.bfloat16),
                            w2_ref[0][c * FC:(c + 1) * FC, :],
                            preferred_element_type=jnp.float32)
    o_ref[...] = (out * w_ref[...]).astype(jnp.bfloat16)


def _combine_kernel(p0_ref, p1_ref, x1_ref, ffn_ref, y_ref):
    col = jax.lax.broadcasted_iota(jnp.int32, (TS, P), 1)
    oh = ((col == p0_ref[...]) | (col == p1_ref[...])).astype(jnp.bfloat16)
    y_ref[...] = x1_ref[...] + jnp.dot(oh, ffn_ref[...],
                                       preferred_element_type=jnp.float32)


def kernel(x, ln1_g, ln1_b, ln2_g, ln2_b, Wq, bq, Wk, bk, Wv, bv, Wo, bo,
           Wr, W1, b1, W2, b2):
    f32 = jnp.float32
    xs2 = x.reshape(S, D)
    row1 = lambda t: t.reshape(1, -1)

    pos_t = jnp.arange(S, dtype=f32)
    div_t = jnp.exp(jnp.arange(0, DH, 2, dtype=f32)
                    * -(jnp.log(10000.0) / DH))
    ang_t = pos_t[:, None] * div_t[None, :]
    sin_t = jnp.sin(ang_t)
    cos_t = jnp.cos(ang_t)

    q, k, v = pl.pallas_call(
        _qkv_kernel,
        grid=(S // TS,),
        in_specs=[
            pl.BlockSpec((TS, D), lambda i: (i, 0)),
            pl.BlockSpec((1, D), lambda i: (0, 0)),
            pl.BlockSpec((1, D), lambda i: (0, 0)),
            pl.BlockSpec((D, D), lambda i: (0, 0)),
            pl.BlockSpec((1, D), lambda i: (0, 0)),
            pl.BlockSpec((D, D), lambda i: (0, 0)),
            pl.BlockSpec((1, D), lambda i: (0, 0)),
            pl.BlockSpec((D, D), lambda i: (0, 0)),
            pl.BlockSpec((1, D), lambda i: (0, 0)),
            pl.BlockSpec((TS, DH // 2), lambda i: (i, 0)),
            pl.BlockSpec((TS, DH // 2), lambda i: (i, 0)),
        ],
        out_specs=[
            pl.BlockSpec((H, TS, DH), lambda i: (0, i, 0)),
            pl.BlockSpec((H, TS, DH), lambda i: (0, i, 0)),
            pl.BlockSpec((H, TS, DH), lambda i: (0, i, 0)),
        ],
        out_shape=[jax.ShapeDtypeStruct((H, S, DH), f32)] * 3,
    )(xs2, row1(ln1_g), row1(ln1_b), Wq, row1(bq), Wk, row1(bk), Wv, row1(bv),
      sin_t, cos_t)

    ao = pl.pallas_call(
        _attn_kernel,
        grid=(H, S // QT),
        in_specs=[
            pl.BlockSpec((1, QT, DH), lambda h, i: (h, i, 0)),
            pl.BlockSpec((1, S, DH), lambda h, i: (h, 0, 0)),
            pl.BlockSpec((1, S, DH), lambda h, i: (h, 0, 0)),
        ],
        out_specs=pl.BlockSpec((1, QT, DH), lambda h, i: (h, i, 0)),
        out_shape=jax.ShapeDtypeStruct((H, S, DH), f32),
    )(q, k, v)

    x1, h2, ti, tw, eu_sum, zl_sum = pl.pallas_call(
        _post_kernel,
        grid=(S // TS,),
        in_specs=[
            pl.BlockSpec((H, TS, DH), lambda i: (0, i, 0)),
            pl.BlockSpec((TS, D), lambda i: (i, 0)),
            pl.BlockSpec((D, D), lambda i: (0, 0)),
            pl.BlockSpec((1, D), lambda i: (0, 0)),
            pl.BlockSpec((1, D), lambda i: (0, 0)),
            pl.BlockSpec((1, D), lambda i: (0, 0)),
            pl.BlockSpec((D, E), lambda i: (0, 0)),
        ],
        out_specs=[
            pl.BlockSpec((TS, D), lambda i: (i, 0)),
            pl.BlockSpec((TS, D), lambda i: (i, 0)),
            pl.BlockSpec((TS, TOPK), lambda i: (i, 0)),
            pl.BlockSpec((TS, TOPK), lambda i: (i, 0)),
            pl.BlockSpec((1, E), lambda i: (0, 0)),
            pl.BlockSpec((1, E), lambda i: (0, 0)),
        ],
        out_shape=[
            jax.ShapeDtypeStruct((S, D), f32),
            jax.ShapeDtypeStruct((S, D), jnp.bfloat16),
            jax.ShapeDtypeStruct((S, TOPK), jnp.int32),
            jax.ShapeDtypeStruct((S, TOPK), f32),
            jax.ShapeDtypeStruct((1, E), f32),
            jax.ShapeDtypeStruct((1, E), f32),
        ],
    )(ao, xs2, Wo, row1(bo), row1(ln2_g), row1(ln2_b), Wr)

    i32 = jnp.int32
    flat_e = ti.reshape(-1)
    flat_w = tw.reshape(-1)
    oh = (flat_e[:, None] == jnp.arange(E, dtype=i32)[None, :]).astype(i32)
    csum = jnp.cumsum(oh, axis=0)
    counts = csum[-1]
    rank = jnp.sum(csum * oh, axis=1) - 1
    padded = ((counts + TM - 1) // TM) * TM
    cum_p = jnp.cumsum(padded)
    gstart = cum_p - padded
    dest = (jnp.sum(oh * gstart[None, :], axis=1) + rank).astype(i32)
    pos_slot = dest
    jj = jnp.arange(S * TOPK, dtype=i32)
    src_tok = jnp.zeros((P,), i32).at[dest].set(jj // TOPK)
    w_pad = jnp.zeros((P,), f32).at[dest].set(flat_w).reshape(P, 1)
    tile_start = jnp.arange(NT, dtype=i32) * TM
    te = jnp.minimum(jnp.searchsorted(cum_p, tile_start, side='right'),
                     E - 1).astype(i32)

    ffn_out = pl.pallas_call(
        _moe_kernel,
        grid_spec=pltpu.PrefetchScalarGridSpec(
            num_scalar_prefetch=1,
            grid=(NT,),
            in_specs=[
                pl.BlockSpec((TM, 1), lambda i, te: (i, 0)),
                pl.BlockSpec((S, D), lambda i, te: (0, 0)),
                pl.BlockSpec((1, D, FFN), lambda i, te: (te[i], 0, 0)),
                pl.BlockSpec((1, 1, FFN), lambda i, te: (te[i], 0, 0)),
                pl.BlockSpec((1, FFN, D), lambda i, te: (te[i], 0, 0)),
                pl.BlockSpec((1, 1, D), lambda i, te: (te[i], 0, 0)),
                pl.BlockSpec((TM, 1), lambda i, te: (i, 0)),
            ],
            out_specs=pl.BlockSpec((TM, D), lambda i, te: (i, 0)),
        ),
        out_shape=jax.ShapeDtypeStruct((P, D), jnp.bfloat16),
    )(te, src_tok.reshape(P, 1), h2, W1.astype(jnp.bfloat16),
      b1.reshape(E, 1, FFN), W2.astype(jnp.bfloat16), b2.reshape(E, 1, D),
      w_pad)

    pos2 = pos_slot.reshape(S, TOPK)
    y = pl.pallas_call(
        _combine_kernel,
        grid=(S // TS,),
        in_specs=[
            pl.BlockSpec((TS, 1), lambda i: (i, 0)),
            pl.BlockSpec((TS, 1), lambda i: (i, 0)),
            pl.BlockSpec((TS, D), lambda i: (i, 0)),
            pl.BlockSpec((P, D), lambda i: (0, 0)),
        ],
        out_specs=pl.BlockSpec((TS, D), lambda i: (i, 0)),
        out_shape=jax.ShapeDtypeStruct((S, D), f32),
    )(pos2[:, 0:1], pos2[:, 1:2], x1, ffn_out)

    eu = eu_sum[0] / jnp.float32(S)
    mu = jnp.mean(eu)
    lb = jnp.mean((eu - mu) ** 2) / (mu * mu + 1e-8) * float(E) * 0.01
    zl = zl_sum[0, 0] / jnp.float32(S) * 0.001
    return y.reshape(B, S, D), lb, zl
